# Initial kernel scaffold; baseline (speedup 1.0000x reference)
import jax
import jax.numpy as jnp
from jax.experimental import pallas as pl


def kernel(x, edge_index, W_in, b_in, W_gat, att_src, att_dst, b_gat, W_res, b_res, ln_g, ln_b, W_out, b_out):
    raise NotImplementedError("write your pallas kernel here")


# speedup vs baseline: 53.9855x; 53.9855x over previous
import dataclasses
import functools

import jax
import jax.numpy as jnp
from jax import lax
from jax.experimental import pallas as pl
from jax.experimental.pallas import tpu as pltpu
from jax.experimental.pallas import tpu_sc as plsc

N = 50000
D_IN = 128
HD = 64
HEADS = 4
F = 16
D_OUT = 512
L = 4

LANES = 16
NSUB = 16
C1 = 1024
C2 = 256
NB = 1000
GRID = N // NB
ACC_ROWS = 50176
DUMP_BLK = 400

_f32 = jnp.float32
_i32 = jnp.int32


def _tc_in_body(x_ref, w_ref, b_ref, o_ref):
    o_ref[...] = jnp.dot(x_ref[...], w_ref[...],
                         preferred_element_type=_f32) + b_ref[...]


def _tc_in(x, w, b):
    return pl.pallas_call(
        _tc_in_body,
        grid=(GRID,),
        in_specs=[
            pl.BlockSpec((NB, D_IN), lambda i: (i, 0)),
            pl.BlockSpec((D_IN, HD), lambda i: (0, 0)),
            pl.BlockSpec((1, HD), lambda i: (0, 0)),
        ],
        out_specs=pl.BlockSpec((NB, HD), lambda i: (i, 0)),
        out_shape=jax.ShapeDtypeStruct((N, HD), _f32),
    )(x, w, b)


def _tc_pre_body(h_ref, wg_ref, wr_ref, br_ref, asm_ref, adm_ref,
                 hh_ref, as_ref, ad_ref, res_ref):
    h = h_ref[...]
    hh = jnp.dot(h, wg_ref[...], preferred_element_type=_f32)
    hh_ref[0] = hh[:, :32]
    hh_ref[1] = hh[:, 32:]
    as_ref[...] = jnp.dot(hh, asm_ref[...], preferred_element_type=_f32)
    ad_ref[...] = jnp.dot(hh, adm_ref[...], preferred_element_type=_f32)
    res_ref[...] = jnp.dot(h, wr_ref[...], preferred_element_type=_f32) \
        + br_ref[...]


def _tc_pre(h, wg, wr, br, asm, adm):
    return pl.pallas_call(
        _tc_pre_body,
        grid=(GRID,),
        in_specs=[
            pl.BlockSpec((NB, HD), lambda i: (i, 0)),
            pl.BlockSpec((HD, HD), lambda i: (0, 0)),
            pl.BlockSpec((HD, HD), lambda i: (0, 0)),
            pl.BlockSpec((1, HD), lambda i: (0, 0)),
            pl.BlockSpec((HD, F), lambda i: (0, 0)),
            pl.BlockSpec((HD, F), lambda i: (0, 0)),
        ],
        out_specs=[
            pl.BlockSpec((2, NB, 32), lambda i: (0, i, 0)),
            pl.BlockSpec((NB, F), lambda i: (i, 0)),
            pl.BlockSpec((NB, F), lambda i: (i, 0)),
            pl.BlockSpec((NB, HD), lambda i: (i, 0)),
        ],
        out_shape=[
            jax.ShapeDtypeStruct((2, N, 32), _f32),
            jax.ShapeDtypeStruct((N, F), _f32),
            jax.ShapeDtypeStruct((N, F), _f32),
            jax.ShapeDtypeStruct((N, HD), _f32),
        ],
    )(h, wg, wr, br, asm, adm)


def _tc_post_body(ma_ref, mb_ref, da_ref, db_ref, res_ref, bg_ref, g_ref,
                  b_ref, h_ref):
    m = jnp.concatenate([ma_ref[...], mb_ref[...]], axis=1)
    d = da_ref[...] + db_ref[...]
    rows_h = lax.broadcasted_iota(_i32, (F, HD), 0)
    cols_h = lax.broadcasted_iota(_i32, (F, HD), 1) // F
    em = jnp.where(rows_h == cols_h, 1.0, 0.0).astype(_f32)
    d64 = jnp.dot(d, em, preferred_element_type=_f32)
    z = m / (d64 + 1e-16) + bg_ref[...] + res_ref[...]
    mu = jnp.mean(z, axis=1, keepdims=True)
    zc = z - mu
    var = jnp.mean(zc * zc, axis=1, keepdims=True)
    zn = zc / jnp.sqrt(var + 1e-5) * g_ref[...] + b_ref[...]
    h_ref[...] = jnp.maximum(zn, 0.0)


def _tc_post(msg_flat, den_flat, res, bg, g, b):
    return pl.pallas_call(
        _tc_post_body,
        grid=(GRID,),
        in_specs=[
            pl.BlockSpec((NB, 32), lambda i: (i, 0)),
            pl.BlockSpec((NB, 32), lambda i: (i + GRID, 0)),
            pl.BlockSpec((NB, F), lambda i: (i, 0)),
            pl.BlockSpec((NB, F), lambda i: (i + GRID, 0)),
            pl.BlockSpec((NB, HD), lambda i: (i, 0)),
            pl.BlockSpec((1, HD), lambda i: (0, 0)),
            pl.BlockSpec((1, HD), lambda i: (0, 0)),
            pl.BlockSpec((1, HD), lambda i: (0, 0)),
        ],
        out_specs=pl.BlockSpec((NB, HD), lambda i: (i, 0)),
        out_shape=jax.ShapeDtypeStruct((N, HD), _f32),
    )(msg_flat, msg_flat, den_flat, den_flat, res, bg, g, b)


def _tc_out_body(h_ref, w_ref, b_ref, o_ref, acc_ref):
    i = pl.program_id(0)

    @pl.when(i == 0)
    def _():
        acc_ref[...] = jnp.zeros((1, HD), _f32)

    acc_ref[...] += jnp.sum(h_ref[...], axis=0, keepdims=True)

    @pl.when(i == GRID - 1)
    def _():
        g = acc_ref[...] * (1.0 / N)
        o_ref[...] = jnp.dot(g, w_ref[...],
                             preferred_element_type=_f32) + b_ref[...]


def _tc_out(h, w, b):
    return pl.pallas_call(
        _tc_out_body,
        grid=(GRID,),
        in_specs=[
            pl.BlockSpec((NB, HD), lambda i: (i, 0)),
            pl.BlockSpec((HD, D_OUT), lambda i: (0, 0)),
            pl.BlockSpec((1, D_OUT), lambda i: (0, 0)),
        ],
        out_specs=pl.BlockSpec((1, D_OUT), lambda i: (0, 0)),
        out_shape=jax.ShapeDtypeStruct((1, D_OUT), _f32),
        scratch_shapes=[pltpu.VMEM((1, HD), _f32)],
    )(h, w, b)


def _sc_compiler_params():
    cp = pltpu.CompilerParams()
    fields = pltpu.CompilerParams.__dataclass_fields__
    if "needs_layout_passes" in fields:
        cp = dataclasses.replace(cp, needs_layout_passes=False)
    if "use_tc_tiling_on_sc" in fields:
        cp = dataclasses.replace(cp, use_tc_tiling_on_sc=False)
    return cp


def _sc_mesh():
    return plsc.VectorSubcoreMesh(core_axis_name="c", subcore_axis_name="s")


def _zero_acc(zb, acc, sub, width):
    nblk = ACC_ROWS // (NSUB * F)

    @pl.loop(0, nblk)
    def _(r):
        pltpu.sync_copy(zb, acc.at[pl.ds((sub * nblk + r) * F, F)])


def _dump_acc(acc, out, core, sub, rows):
    nblk = rows // DUMP_BLK

    @pl.loop(0, (nblk + NSUB - 1) // NSUB)
    def _(r):
        blk = r * NSUB + sub

        @pl.when(blk < nblk)
        def _():
            pltpu.sync_copy(
                acc.at[pl.ds(blk * DUMP_BLK, DUMP_BLK)],
                out.at[pl.ds(core * rows + blk * DUMP_BLK, DUMP_BLK)])


def _sc_den_kernel(ep):
    nchunks = ep // C1

    @functools.partial(
        pl.kernel,
        compiler_params=_sc_compiler_params(),
        out_type=[
            jax.ShapeDtypeStruct((4 * ep,), _f32),
            jax.ShapeDtypeStruct((2 * N, F), _f32),
        ],
        mesh=_sc_mesh(),
        scratch_types=[
            pltpu.VMEM((C1,), _i32),
            pltpu.VMEM((C1,), _i32),
            pltpu.VMEM((C1,), _i32),
            pltpu.VMEM((C1, F), _f32),
            pltpu.VMEM((C1, F), _f32),
            pltpu.VMEM((C1, F), _f32),
            pltpu.VMEM((4 * C1,), _f32),
            pltpu.VMEM((F, F), _f32),
            pltpu.VMEM_SHARED((ACC_ROWS, F), _f32),
        ],
    )
    def k(as_hbm, ad_hbm, src_hbm, dst_hbm, ex_out, den_out,
          src_v, dst_v, dcl_v, asr, adr, denr, exp_v, zb, den_acc):
        core = lax.axis_index("c")
        sub = lax.axis_index("s")
        zero16 = jnp.zeros((LANES,), _f32)
        iota = lax.iota(_i32, LANES)
        headmask = iota < HEADS
        w = sub * 2 + core

        @pl.loop(0, F)
        def _(i):
            zb[i, pl.ds(0, LANES)] = zero16

        _zero_acc(zb, den_acc, sub, F)
        plsc.subcore_barrier()

        @pl.loop(0, nchunks // 32)
        def _(jj):
            base = (jj * 32 + w) * C1
            pltpu.sync_copy(src_hbm.at[pl.ds(base, C1)], src_v)
            pltpu.sync_copy(dst_hbm.at[pl.ds(base, C1)], dst_v)

            @pl.loop(0, C1 // LANES)
            def _(i):
                o = i * LANES
                dv = dst_v[pl.ds(o, LANES)]
                dcl_v[pl.ds(o, LANES)] = jnp.minimum(dv, N - 1)

            pltpu.sync_copy(as_hbm.at[src_v], asr)
            pltpu.sync_copy(ad_hbm.at[dcl_v], adr)

            @pl.loop(0, C1)
            def _(e):
                s = asr[e, pl.ds(0, LANES)] + adr[e, pl.ds(0, LANES)]
                s = jnp.maximum(s, 0.2 * s)
                ex = jnp.exp(s)
                denr[e, pl.ds(0, LANES)] = jnp.where(headmask, ex, 0.0)
                plsc.store_scatter(exp_v, [iota + 4 * e], ex, mask=headmask)

            pltpu.sync_copy(denr, den_acc.at[dst_v], add=True)
            pltpu.sync_copy(exp_v, ex_out.at[pl.ds(4 * base, 4 * C1)])

        plsc.subcore_barrier()
        _dump_acc(den_acc, den_out, core, sub, N)

    return k


def _sc_msg_kernel(ep):
    nchunks = ep // C2

    @functools.partial(
        pl.kernel,
        compiler_params=_sc_compiler_params(),
        out_type=jax.ShapeDtypeStruct((2 * N, 32), _f32),
        mesh=_sc_mesh(),
        scratch_types=[
            pltpu.VMEM((C2,), _i32),
            pltpu.VMEM((C2,), _i32),
            pltpu.VMEM((C2,), _i32),
            pltpu.VMEM((4 * C2,), _f32),
            pltpu.VMEM((C2, 32), _f32),
            pltpu.VMEM((C2, 32), _f32),
            pltpu.VMEM((F, 32), _f32),
            pltpu.VMEM_SHARED((ACC_ROWS, 32), _f32),
        ],
    )
    def k(hh_hbm, ex_hbm, src_hbm, dst_hbm, msg_out,
          src_v, dst_v, soff_v, ex_v, hr, wr, zb, msg_acc):
        core = lax.axis_index("c")
        sub = lax.axis_index("s")
        zero16 = jnp.zeros((LANES,), _f32)
        h0idx = jnp.full((LANES,), 2 * core, _i32)
        h1idx = jnp.full((LANES,), 2 * core + 1, _i32)

        @pl.loop(0, F)
        def _(i):
            zb[i, pl.ds(0, LANES)] = zero16
            zb[i, pl.ds(LANES, LANES)] = zero16

        _zero_acc(zb, msg_acc, sub, 32)
        plsc.subcore_barrier()

        @pl.loop(0, nchunks // NSUB)
        def _(jj):
            base = (jj * NSUB + sub) * C2
            pltpu.sync_copy(src_hbm.at[pl.ds(base, C2)], src_v)
            pltpu.sync_copy(dst_hbm.at[pl.ds(base, C2)], dst_v)
            pltpu.sync_copy(ex_hbm.at[pl.ds(4 * base, 4 * C2)], ex_v)

            @pl.loop(0, C2 // LANES)
            def _(i):
                o = i * LANES
                soff_v[pl.ds(o, LANES)] = src_v[pl.ds(o, LANES)] + core * N

            pltpu.sync_copy(hh_hbm.at[soff_v], hr)

            @pl.loop(0, C2)
            def _(e):
                sp0 = plsc.load_gather(ex_v, [h0idx + 4 * e])
                sp1 = plsc.load_gather(ex_v, [h1idx + 4 * e])
                wr[e, pl.ds(0, LANES)] = hr[e, pl.ds(0, LANES)] * sp0
                wr[e, pl.ds(LANES, LANES)] = hr[e, pl.ds(LANES, LANES)] * sp1

            pltpu.sync_copy(wr, msg_acc.at[dst_v], add=True)

        plsc.subcore_barrier()
        _dump_acc(msg_acc, msg_out, core, sub, N)

    return k


def _att_expand(att_l):
    att_flat = att_l.reshape(HD)
    cols = jnp.arange(F, dtype=_i32)[None, :]
    rows_h = (jnp.arange(HD, dtype=_i32) // F)[:, None]
    return jnp.where(cols == rows_h, att_flat[:, None], 0.0).astype(_f32)


def kernel(x, edge_index, W_in, b_in, W_gat, att_src, att_dst, b_gat,
           W_res, b_res, ln_g, ln_b, W_out, b_out):
    e_total = edge_index.shape[1] + N
    align = 32 * C1
    ep = ((e_total + align - 1) // align) * align
    pad = ep - e_total
    loops = jnp.arange(N, dtype=_i32)
    src = jnp.concatenate(
        [edge_index[0].astype(_i32), loops, jnp.zeros((pad,), _i32)])
    dst = jnp.concatenate(
        [edge_index[1].astype(_i32), loops, jnp.full((pad,), N, _i32)])

    sc_den = _sc_den_kernel(ep)
    sc_msg = _sc_msg_kernel(ep)

    h = _tc_in(x, W_in, b_in.reshape(1, HD))
    for l in range(L):
        asm = _att_expand(att_src[l])
        adm = _att_expand(att_dst[l])
        hh2, as_t, ad_t, res = _tc_pre(
            h, W_gat[l], W_res[l], b_res[l].reshape(1, HD), asm, adm)
        ex, den_flat = sc_den(as_t, ad_t, src, dst)
        msg_flat = sc_msg(hh2.reshape(2 * N, 32), ex, src, dst)
        h = _tc_post(msg_flat, den_flat, res, b_gat[l].reshape(1, HD),
                     ln_g[l].reshape(1, HD), ln_b[l].reshape(1, HD))
    return _tc_out(h, W_out, b_out.reshape(1, D_OUT))


# speedup vs baseline: 86.5095x; 1.6025x over previous
import dataclasses
import functools

import jax
import jax.numpy as jnp
from jax import lax
from jax.experimental import pallas as pl
from jax.experimental.pallas import tpu as pltpu
from jax.experimental.pallas import tpu_sc as plsc

N = 50000
D_IN = 128
HD = 64
HEADS = 4
F = 16
D_OUT = 512
L = 4

LANES = 16
NSUB = 16
C1 = 1024
C2 = 256
NB = 1000
GRID = N // NB
ACC_ROWS = 50176
DUMP_BLK = 400

_f32 = jnp.float32
_i32 = jnp.int32


def _tc_in_body(x_ref, w_ref, b_ref, o_ref):
    o_ref[...] = jnp.dot(x_ref[...], w_ref[...],
                         preferred_element_type=_f32) + b_ref[...]


def _tc_in(x, w, b):
    return pl.pallas_call(
        _tc_in_body,
        grid=(GRID,),
        in_specs=[
            pl.BlockSpec((NB, D_IN), lambda i: (i, 0)),
            pl.BlockSpec((D_IN, HD), lambda i: (0, 0)),
            pl.BlockSpec((1, HD), lambda i: (0, 0)),
        ],
        out_specs=pl.BlockSpec((NB, HD), lambda i: (i, 0)),
        out_shape=jax.ShapeDtypeStruct((N, HD), _f32),
    )(x, w, b)


def _tc_pre_body(h_ref, wg_ref, wr_ref, br_ref, asm_ref, adm_ref,
                 hh_ref, as_ref, ad_ref, res_ref):
    h = h_ref[...]
    hh = jnp.dot(h, wg_ref[...], preferred_element_type=_f32)
    hh_ref[0] = hh[:, :32]
    hh_ref[1] = hh[:, 32:]
    as_ref[...] = jnp.dot(hh, asm_ref[...], preferred_element_type=_f32)
    ad_ref[...] = jnp.dot(hh, adm_ref[...], preferred_element_type=_f32)
    res_ref[...] = jnp.dot(h, wr_ref[...], preferred_element_type=_f32) \
        + br_ref[...]


def _tc_pre(h, wg, wr, br, asm, adm):
    return pl.pallas_call(
        _tc_pre_body,
        grid=(GRID,),
        in_specs=[
            pl.BlockSpec((NB, HD), lambda i: (i, 0)),
            pl.BlockSpec((HD, HD), lambda i: (0, 0)),
            pl.BlockSpec((HD, HD), lambda i: (0, 0)),
            pl.BlockSpec((1, HD), lambda i: (0, 0)),
            pl.BlockSpec((HD, F), lambda i: (0, 0)),
            pl.BlockSpec((HD, F), lambda i: (0, 0)),
        ],
        out_specs=[
            pl.BlockSpec((2, NB, 32), lambda i: (0, i, 0)),
            pl.BlockSpec((NB, F), lambda i: (i, 0)),
            pl.BlockSpec((NB, F), lambda i: (i, 0)),
            pl.BlockSpec((NB, HD), lambda i: (i, 0)),
        ],
        out_shape=[
            jax.ShapeDtypeStruct((2, N, 32), _f32),
            jax.ShapeDtypeStruct((N, F), _f32),
            jax.ShapeDtypeStruct((N, F), _f32),
            jax.ShapeDtypeStruct((N, HD), _f32),
        ],
    )(h, wg, wr, br, asm, adm)


def _tc_post_body(ma_ref, mb_ref, da_ref, db_ref, res_ref, bg_ref, g_ref,
                  b_ref, h_ref):
    m = jnp.concatenate([ma_ref[...], mb_ref[...]], axis=1)
    d = da_ref[...] + db_ref[...]
    rows_h = lax.broadcasted_iota(_i32, (F, HD), 0)
    cols_h = lax.broadcasted_iota(_i32, (F, HD), 1) // F
    em = jnp.where(rows_h == cols_h, 1.0, 0.0).astype(_f32)
    d64 = jnp.dot(d, em, preferred_element_type=_f32)
    z = m / (d64 + 1e-16) + bg_ref[...] + res_ref[...]
    mu = jnp.mean(z, axis=1, keepdims=True)
    zc = z - mu
    var = jnp.mean(zc * zc, axis=1, keepdims=True)
    zn = zc / jnp.sqrt(var + 1e-5) * g_ref[...] + b_ref[...]
    h_ref[...] = jnp.maximum(zn, 0.0)


def _tc_post(msg_flat, den_flat, res, bg, g, b):
    return pl.pallas_call(
        _tc_post_body,
        grid=(GRID,),
        in_specs=[
            pl.BlockSpec((NB, 32), lambda i: (i, 0)),
            pl.BlockSpec((NB, 32), lambda i: (i + GRID, 0)),
            pl.BlockSpec((NB, F), lambda i: (i, 0)),
            pl.BlockSpec((NB, F), lambda i: (i + GRID, 0)),
            pl.BlockSpec((NB, HD), lambda i: (i, 0)),
            pl.BlockSpec((1, HD), lambda i: (0, 0)),
            pl.BlockSpec((1, HD), lambda i: (0, 0)),
            pl.BlockSpec((1, HD), lambda i: (0, 0)),
        ],
        out_specs=pl.BlockSpec((NB, HD), lambda i: (i, 0)),
        out_shape=jax.ShapeDtypeStruct((N, HD), _f32),
    )(msg_flat, msg_flat, den_flat, den_flat, res, bg, g, b)


def _tc_out_body(h_ref, w_ref, b_ref, o_ref, acc_ref):
    i = pl.program_id(0)

    @pl.when(i == 0)
    def _():
        acc_ref[...] = jnp.zeros((1, HD), _f32)

    acc_ref[...] += jnp.sum(h_ref[...], axis=0, keepdims=True)

    @pl.when(i == GRID - 1)
    def _():
        g = acc_ref[...] * (1.0 / N)
        o_ref[...] = jnp.dot(g, w_ref[...],
                             preferred_element_type=_f32) + b_ref[...]


def _tc_out(h, w, b):
    return pl.pallas_call(
        _tc_out_body,
        grid=(GRID,),
        in_specs=[
            pl.BlockSpec((NB, HD), lambda i: (i, 0)),
            pl.BlockSpec((HD, D_OUT), lambda i: (0, 0)),
            pl.BlockSpec((1, D_OUT), lambda i: (0, 0)),
        ],
        out_specs=pl.BlockSpec((1, D_OUT), lambda i: (0, 0)),
        out_shape=jax.ShapeDtypeStruct((1, D_OUT), _f32),
        scratch_shapes=[pltpu.VMEM((1, HD), _f32)],
    )(h, w, b)


def _sc_compiler_params():
    cp = pltpu.CompilerParams()
    fields = pltpu.CompilerParams.__dataclass_fields__
    if "needs_layout_passes" in fields:
        cp = dataclasses.replace(cp, needs_layout_passes=False)
    if "use_tc_tiling_on_sc" in fields:
        cp = dataclasses.replace(cp, use_tc_tiling_on_sc=False)
    return cp


def _sc_mesh():
    return plsc.VectorSubcoreMesh(core_axis_name="c", subcore_axis_name="s")


def _zero_acc(zb, acc, sub, width):
    nblk = ACC_ROWS // (NSUB * F)

    @pl.loop(0, nblk)
    def _(r):
        pltpu.sync_copy(zb, acc.at[pl.ds((sub * nblk + r) * F, F)])


def _dump_acc(acc, out, core, sub, rows):
    nblk = rows // DUMP_BLK

    @pl.loop(0, (nblk + NSUB - 1) // NSUB)
    def _(r):
        blk = r * NSUB + sub

        @pl.when(blk < nblk)
        def _():
            pltpu.sync_copy(
                acc.at[pl.ds(blk * DUMP_BLK, DUMP_BLK)],
                out.at[pl.ds(core * rows + blk * DUMP_BLK, DUMP_BLK)])


def _sc_den_kernel(ep):
    nchunks = ep // C1

    @functools.partial(
        pl.kernel,
        compiler_params=_sc_compiler_params(),
        out_type=[
            jax.ShapeDtypeStruct((4 * ep,), _f32),
            jax.ShapeDtypeStruct((2 * N, F), _f32),
        ],
        mesh=_sc_mesh(),
        scratch_types=[
            pltpu.VMEM((C1,), _i32),
            pltpu.VMEM((C1,), _i32),
            pltpu.VMEM((C1,), _i32),
            pltpu.VMEM((C1, F), _f32),
            pltpu.VMEM((C1, F), _f32),
            pltpu.VMEM((C1, F), _f32),
            pltpu.VMEM((4 * C1,), _f32),
            pltpu.VMEM((F, F), _f32),
            pltpu.VMEM_SHARED((ACC_ROWS, F), _f32),
        ],
    )
    def k(as_hbm, ad_hbm, src_hbm, dst_hbm, ex_out, den_out,
          src_v, dst_v, dcl_v, asr, adr, denr, exp_v, zb, den_acc):
        core = lax.axis_index("c")
        sub = lax.axis_index("s")
        zero16 = jnp.zeros((LANES,), _f32)
        iota = lax.iota(_i32, LANES)
        headmask = iota < HEADS
        w = sub * 2 + core

        @pl.loop(0, F)
        def _(i):
            zb[i, pl.ds(0, LANES)] = zero16

        _zero_acc(zb, den_acc, sub, F)
        plsc.subcore_barrier()

        @pl.loop(0, nchunks // 32)
        def _(jj):
            base = (jj * 32 + w) * C1
            pltpu.sync_copy(src_hbm.at[pl.ds(base, C1)], src_v)
            pltpu.sync_copy(dst_hbm.at[pl.ds(base, C1)], dst_v)

            @plsc.parallel_loop(0, C1 // LANES, unroll=4)
            def _(i):
                o = i * LANES
                dv = dst_v[pl.ds(o, LANES)]
                dcl_v[pl.ds(o, LANES)] = jnp.minimum(dv, N - 1)

            pltpu.sync_copy(as_hbm.at[src_v], asr)
            pltpu.sync_copy(ad_hbm.at[dcl_v], adr)

            @plsc.parallel_loop(0, C1, unroll=8)
            def _(e):
                s = asr[e, pl.ds(0, LANES)] + adr[e, pl.ds(0, LANES)]
                s = jnp.maximum(s, 0.2 * s)
                ex = jnp.exp(s)
                denr[e, pl.ds(0, LANES)] = jnp.where(headmask, ex, 0.0)
                plsc.store_scatter(exp_v, [iota + 4 * e], ex, mask=headmask)

            pltpu.sync_copy(denr, den_acc.at[dst_v], add=True)
            pltpu.sync_copy(exp_v, ex_out.at[pl.ds(4 * base, 4 * C1)])

        plsc.subcore_barrier()
        _dump_acc(den_acc, den_out, core, sub, N)

    return k


def _sc_msg_kernel(ep):
    nchunks = ep // C2

    @functools.partial(
        pl.kernel,
        compiler_params=_sc_compiler_params(),
        out_type=jax.ShapeDtypeStruct((2 * N, 32), _f32),
        mesh=_sc_mesh(),
        scratch_types=[
            pltpu.VMEM((C2,), _i32),
            pltpu.VMEM((C2,), _i32),
            pltpu.VMEM((C2,), _i32),
            pltpu.VMEM((4 * C2,), _f32),
            pltpu.VMEM((C2, 32), _f32),
            pltpu.VMEM((C2, 32), _f32),
            pltpu.VMEM((F, 32), _f32),
            pltpu.VMEM_SHARED((ACC_ROWS, 32), _f32),
        ],
    )
    def k(hh_hbm, ex_hbm, src_hbm, dst_hbm, msg_out,
          src_v, dst_v, soff_v, ex_v, hr, wr, zb, msg_acc):
        core = lax.axis_index("c")
        sub = lax.axis_index("s")
        zero16 = jnp.zeros((LANES,), _f32)
        h0idx = jnp.full((LANES,), 2 * core, _i32)
        h1idx = jnp.full((LANES,), 2 * core + 1, _i32)

        @pl.loop(0, F)
        def _(i):
            zb[i, pl.ds(0, LANES)] = zero16
            zb[i, pl.ds(LANES, LANES)] = zero16

        _zero_acc(zb, msg_acc, sub, 32)
        plsc.subcore_barrier()

        @pl.loop(0, nchunks // NSUB)
        def _(jj):
            base = (jj * NSUB + sub) * C2
            pltpu.sync_copy(src_hbm.at[pl.ds(base, C2)], src_v)
            pltpu.sync_copy(dst_hbm.at[pl.ds(base, C2)], dst_v)
            pltpu.sync_copy(ex_hbm.at[pl.ds(4 * base, 4 * C2)], ex_v)

            @plsc.parallel_loop(0, C2 // LANES, unroll=4)
            def _(i):
                o = i * LANES
                soff_v[pl.ds(o, LANES)] = src_v[pl.ds(o, LANES)] + core * N

            pltpu.sync_copy(hh_hbm.at[soff_v], hr)

            @plsc.parallel_loop(0, C2, unroll=8)
            def _(e):
                sp0 = plsc.load_gather(ex_v, [h0idx + 4 * e])
                sp1 = plsc.load_gather(ex_v, [h1idx + 4 * e])
                wr[e, pl.ds(0, LANES)] = hr[e, pl.ds(0, LANES)] * sp0
                wr[e, pl.ds(LANES, LANES)] = hr[e, pl.ds(LANES, LANES)] * sp1

            pltpu.sync_copy(wr, msg_acc.at[dst_v], add=True)

        plsc.subcore_barrier()
        _dump_acc(msg_acc, msg_out, core, sub, N)

    return k


def _att_expand(att_l):
    att_flat = att_l.reshape(HD)
    cols = jnp.arange(F, dtype=_i32)[None, :]
    rows_h = (jnp.arange(HD, dtype=_i32) // F)[:, None]
    return jnp.where(cols == rows_h, att_flat[:, None], 0.0).astype(_f32)


def kernel(x, edge_index, W_in, b_in, W_gat, att_src, att_dst, b_gat,
           W_res, b_res, ln_g, ln_b, W_out, b_out):
    e_total = edge_index.shape[1] + N
    align = 32 * C1
    ep = ((e_total + align - 1) // align) * align
    pad = ep - e_total
    loops = jnp.arange(N, dtype=_i32)
    src = jnp.concatenate(
        [edge_index[0].astype(_i32), loops, jnp.zeros((pad,), _i32)])
    dst = jnp.concatenate(
        [edge_index[1].astype(_i32), loops, jnp.full((pad,), N, _i32)])

    sc_den = _sc_den_kernel(ep)
    sc_msg = _sc_msg_kernel(ep)

    h = _tc_in(x, W_in, b_in.reshape(1, HD))
    for l in range(L):
        asm = _att_expand(att_src[l])
        adm = _att_expand(att_dst[l])
        hh2, as_t, ad_t, res = _tc_pre(
            h, W_gat[l], W_res[l], b_res[l].reshape(1, HD), asm, adm)
        ex, den_flat = sc_den(as_t, ad_t, src, dst)
        msg_flat = sc_msg(hh2.reshape(2 * N, 32), ex, src, dst)
        h = _tc_post(msg_flat, den_flat, res, b_gat[l].reshape(1, HD),
                     ln_g[l].reshape(1, HD), ln_b[l].reshape(1, HD))
    return _tc_out(h, W_out, b_out.reshape(1, D_OUT))


# speedup vs baseline: 145.2268x; 1.6787x over previous
import dataclasses
import functools

import jax
import jax.numpy as jnp
from jax import lax
from jax.experimental import pallas as pl
from jax.experimental.pallas import tpu as pltpu
from jax.experimental.pallas import tpu_sc as plsc

N = 50000
D_IN = 128
HD = 64
HEADS = 4
F = 16
D_OUT = 512
L = 4

LANES = 16
NSUB = 16
C1 = 512
C2 = 256
NB = 1000
GRID = N // NB
ACC_ROWS = 50176
DUMP_BLK = 400

_f32 = jnp.float32
_i32 = jnp.int32


def _tc_in_body(x_ref, w_ref, b_ref, o_ref):
    o_ref[...] = jnp.dot(x_ref[...], w_ref[...],
                         preferred_element_type=_f32) + b_ref[...]


def _tc_in(x, w, b):
    return pl.pallas_call(
        _tc_in_body,
        grid=(GRID,),
        in_specs=[
            pl.BlockSpec((NB, D_IN), lambda i: (i, 0)),
            pl.BlockSpec((D_IN, HD), lambda i: (0, 0)),
            pl.BlockSpec((1, HD), lambda i: (0, 0)),
        ],
        out_specs=pl.BlockSpec((NB, HD), lambda i: (i, 0)),
        out_shape=jax.ShapeDtypeStruct((N, HD), _f32),
    )(x, w, b)


def _tc_pre_body(h_ref, wg_ref, wr_ref, br_ref, asm_ref, adm_ref,
                 hh_ref, as_ref, ad_ref, res_ref):
    h = h_ref[...]
    hh = jnp.dot(h, wg_ref[...], preferred_element_type=_f32)
    hh_ref[0] = hh[:, :32]
    hh_ref[1] = hh[:, 32:]
    as_ref[...] = jnp.dot(hh, asm_ref[...], preferred_element_type=_f32)
    ad_ref[...] = jnp.dot(hh, adm_ref[...], preferred_element_type=_f32)
    res_ref[...] = jnp.dot(h, wr_ref[...], preferred_element_type=_f32) \
        + br_ref[...]


def _tc_pre(h, wg, wr, br, asm, adm):
    return pl.pallas_call(
        _tc_pre_body,
        grid=(GRID,),
        in_specs=[
            pl.BlockSpec((NB, HD), lambda i: (i, 0)),
            pl.BlockSpec((HD, HD), lambda i: (0, 0)),
            pl.BlockSpec((HD, HD), lambda i: (0, 0)),
            pl.BlockSpec((1, HD), lambda i: (0, 0)),
            pl.BlockSpec((HD, F), lambda i: (0, 0)),
            pl.BlockSpec((HD, F), lambda i: (0, 0)),
        ],
        out_specs=[
            pl.BlockSpec((2, NB, 32), lambda i: (0, i, 0)),
            pl.BlockSpec((NB, F), lambda i: (i, 0)),
            pl.BlockSpec((NB, F), lambda i: (i, 0)),
            pl.BlockSpec((NB, HD), lambda i: (i, 0)),
        ],
        out_shape=[
            jax.ShapeDtypeStruct((2, N, 32), _f32),
            jax.ShapeDtypeStruct((N, F), _f32),
            jax.ShapeDtypeStruct((N, F), _f32),
            jax.ShapeDtypeStruct((N, HD), _f32),
        ],
    )(h, wg, wr, br, asm, adm)


def _tc_post_body(ma_ref, mb_ref, da_ref, db_ref, res_ref, bg_ref, g_ref,
                  b_ref, h_ref):
    m = jnp.concatenate([ma_ref[...], mb_ref[...]], axis=1)
    d = da_ref[...] + db_ref[...]
    rows_h = lax.broadcasted_iota(_i32, (F, HD), 0)
    cols_h = lax.broadcasted_iota(_i32, (F, HD), 1) // F
    em = jnp.where(rows_h == cols_h, 1.0, 0.0).astype(_f32)
    d64 = jnp.dot(d, em, preferred_element_type=_f32)
    z = m / (d64 + 1e-16) + bg_ref[...] + res_ref[...]
    mu = jnp.mean(z, axis=1, keepdims=True)
    zc = z - mu
    var = jnp.mean(zc * zc, axis=1, keepdims=True)
    zn = zc / jnp.sqrt(var + 1e-5) * g_ref[...] + b_ref[...]
    h_ref[...] = jnp.maximum(zn, 0.0)


def _tc_post(msg_flat, den_flat, res, bg, g, b):
    return pl.pallas_call(
        _tc_post_body,
        grid=(GRID,),
        in_specs=[
            pl.BlockSpec((NB, 32), lambda i: (i, 0)),
            pl.BlockSpec((NB, 32), lambda i: (i + GRID, 0)),
            pl.BlockSpec((NB, F), lambda i: (i, 0)),
            pl.BlockSpec((NB, F), lambda i: (i + GRID, 0)),
            pl.BlockSpec((NB, HD), lambda i: (i, 0)),
            pl.BlockSpec((1, HD), lambda i: (0, 0)),
            pl.BlockSpec((1, HD), lambda i: (0, 0)),
            pl.BlockSpec((1, HD), lambda i: (0, 0)),
        ],
        out_specs=pl.BlockSpec((NB, HD), lambda i: (i, 0)),
        out_shape=jax.ShapeDtypeStruct((N, HD), _f32),
    )(msg_flat, msg_flat, den_flat, den_flat, res, bg, g, b)


def _tc_out_body(h_ref, w_ref, b_ref, o_ref, acc_ref):
    i = pl.program_id(0)

    @pl.when(i == 0)
    def _():
        acc_ref[...] = jnp.zeros((1, HD), _f32)

    acc_ref[...] += jnp.sum(h_ref[...], axis=0, keepdims=True)

    @pl.when(i == GRID - 1)
    def _():
        g = acc_ref[...] * (1.0 / N)
        o_ref[...] = jnp.dot(g, w_ref[...],
                             preferred_element_type=_f32) + b_ref[...]


def _tc_out(h, w, b):
    return pl.pallas_call(
        _tc_out_body,
        grid=(GRID,),
        in_specs=[
            pl.BlockSpec((NB, HD), lambda i: (i, 0)),
            pl.BlockSpec((HD, D_OUT), lambda i: (0, 0)),
            pl.BlockSpec((1, D_OUT), lambda i: (0, 0)),
        ],
        out_specs=pl.BlockSpec((1, D_OUT), lambda i: (0, 0)),
        out_shape=jax.ShapeDtypeStruct((1, D_OUT), _f32),
        scratch_shapes=[pltpu.VMEM((1, HD), _f32)],
    )(h, w, b)


def _sc_compiler_params():
    cp = pltpu.CompilerParams()
    fields = pltpu.CompilerParams.__dataclass_fields__
    if "needs_layout_passes" in fields:
        cp = dataclasses.replace(cp, needs_layout_passes=False)
    if "use_tc_tiling_on_sc" in fields:
        cp = dataclasses.replace(cp, use_tc_tiling_on_sc=False)
    return cp


def _sc_mesh():
    return plsc.VectorSubcoreMesh(core_axis_name="c", subcore_axis_name="s")


def _zero_acc(zrows, acc, sub):
    rows = ACC_ROWS // NSUB
    zc = zrows.shape[0]

    @pl.loop(0, rows // zc)
    def _(r):
        pltpu.sync_copy(zrows, acc.at[pl.ds(sub * rows + r * zc, zc)])

    rem = rows % zc
    if rem:
        pltpu.sync_copy(zrows.at[pl.ds(0, rem)],
                        acc.at[pl.ds(sub * rows + (rows // zc) * zc, rem)])


def _dump_acc(acc, out, core, sub, rows):
    nblk = rows // DUMP_BLK

    @pl.loop(0, (nblk + NSUB - 1) // NSUB)
    def _(r):
        blk = r * NSUB + sub

        @pl.when(blk < nblk)
        def _():
            pltpu.sync_copy(
                acc.at[pl.ds(blk * DUMP_BLK, DUMP_BLK)],
                out.at[pl.ds(core * rows + blk * DUMP_BLK, DUMP_BLK)])


def _sc_den_kernel(ep):
    nch = ep // (32 * C1)

    @functools.partial(
        pl.kernel,
        compiler_params=_sc_compiler_params(),
        out_type=[
            jax.ShapeDtypeStruct((4 * ep,), _f32),
            jax.ShapeDtypeStruct((2 * N, F), _f32),
        ],
        mesh=_sc_mesh(),
        scratch_types=[
            pltpu.VMEM((2, C1), _i32),
            pltpu.VMEM((2, C1), _i32),
            pltpu.VMEM((2, C1), _i32),
            pltpu.VMEM((2, C1, F), _f32),
            pltpu.VMEM((2, C1, F), _f32),
            pltpu.VMEM((C1, F), _f32),
            pltpu.VMEM((2, 4 * C1), _f32),
        ] + [pltpu.SemaphoreType.DMA] * 10 + [
            pltpu.VMEM_SHARED((ACC_ROWS, F), _f32),
        ],
    )
    def k(as_hbm, ad_hbm, src_hbm, dst_hbm, ex_out, den_out,
          srcs, dsts, dcls, asrs, adrs, denr, exps,
          ss0, ss1, sd0, sd1, sa0, sa1, sb0, sb1, sw0, sw1, den_acc):
        core = lax.axis_index("c")
        sub = lax.axis_index("s")
        s_src, s_dst = (ss0, ss1), (sd0, sd1)
        s_as, s_ad, s_wr = (sa0, sa1), (sb0, sb1), (sw0, sw1)
        zero16 = jnp.zeros((LANES,), _f32)
        iota = lax.iota(_i32, LANES)
        headmask = iota < HEADS
        w = sub * 2 + core

        @plsc.parallel_loop(0, C1, unroll=8)
        def _(e):
            denr[e, pl.ds(0, LANES)] = zero16

        _zero_acc(denr, den_acc, sub)
        plsc.subcore_barrier()

        def idx_copies(t, b):
            base = (t * 32 + w) * C1
            return (
                pltpu.make_async_copy(
                    src_hbm.at[pl.ds(base, C1)], srcs.at[b], s_src[b]),
                pltpu.make_async_copy(
                    dst_hbm.at[pl.ds(base, C1)], dsts.at[b], s_dst[b]),
            )

        def gather_copies(b):
            return (
                pltpu.make_async_copy(as_hbm.at[srcs.at[b]], asrs.at[b],
                                      s_as[b]),
                pltpu.make_async_copy(ad_hbm.at[dcls.at[b]], adrs.at[b],
                                      s_ad[b]),
            )

        def prep_and_gather(b):
            @plsc.parallel_loop(0, C1 // LANES, unroll=4)
            def _(i):
                o = i * LANES
                dcls[b, pl.ds(o, LANES)] = \
                    jnp.minimum(dsts[b, pl.ds(o, LANES)], N - 1)

            for d in gather_copies(b):
                d.start()

        def ex_write(t, b):
            base = (t * 32 + w) * C1
            return pltpu.make_async_copy(
                exps.at[b], ex_out.at[pl.ds(4 * base, 4 * C1)], s_wr[b])

        for d in idx_copies(0, 0):
            d.start()
        for d in idx_copies(0, 0):
            d.wait()
        prep_and_gather(0)
        for d in idx_copies(1, 1):
            d.start()

        @pl.loop(0, nch // 2)
        def _(to):
            for b in (0, 1):
                t = to * 2 + b
                q = 1 - b

                @pl.when(t + 1 < nch)
                def _():
                    for d in idx_copies(t + 1, q):
                        d.wait()
                    prep_and_gather(q)

                for d in gather_copies(b):
                    d.wait()

                @pl.when(t >= 2)
                def _():
                    ex_write(t - 2, b).wait()

                @plsc.parallel_loop(0, C1, unroll=8)
                def _(e):
                    s = asrs[b, e, pl.ds(0, LANES)] + adrs[b, e, pl.ds(0, LANES)]
                    s = jnp.maximum(s, 0.2 * s)
                    ex = jnp.exp(s)
                    denr[e, pl.ds(0, LANES)] = jnp.where(headmask, ex, 0.0)
                    plsc.store_scatter(exps.at[b], [iota + 4 * e], ex,
                                       mask=headmask)

                pltpu.sync_copy(denr, den_acc.at[dsts.at[b]], add=True)
                ex_write(t, b).start()

                @pl.when(t + 2 < nch)
                def _():
                    for d in idx_copies(t + 2, b):
                        d.start()

        for b in (0, 1):
            ex_write(nch - 2 + b, b).wait()

        plsc.subcore_barrier()
        _dump_acc(den_acc, den_out, core, sub, N)

    return k


def _sc_msg_kernel(ep):
    nch = ep // (NSUB * C2)

    @functools.partial(
        pl.kernel,
        compiler_params=_sc_compiler_params(),
        out_type=jax.ShapeDtypeStruct((2 * N, 32), _f32),
        mesh=_sc_mesh(),
        scratch_types=[
            pltpu.VMEM((2, C2), _i32),
            pltpu.VMEM((2, C2), _i32),
            pltpu.VMEM((2, C2), _i32),
            pltpu.VMEM((2, 4 * C2), _f32),
            pltpu.VMEM((2, C2, 32), _f32),
            pltpu.VMEM((C2, 32), _f32),
        ] + [pltpu.SemaphoreType.DMA] * 8 + [
            pltpu.VMEM_SHARED((ACC_ROWS, 32), _f32),
        ],
    )
    def k(hh_hbm, ex_hbm, src_hbm, dst_hbm, msg_out,
          srcs, dsts, soffs, exvs, hrs, wr,
          ss0, ss1, sd0, sd1, se0, se1, sh0, sh1, msg_acc):
        core = lax.axis_index("c")
        sub = lax.axis_index("s")
        s_src, s_dst = (ss0, ss1), (sd0, sd1)
        s_ex, s_hr = (se0, se1), (sh0, sh1)
        zero16 = jnp.zeros((LANES,), _f32)
        h0idx = jnp.full((LANES,), 2 * core, _i32)
        h1idx = jnp.full((LANES,), 2 * core + 1, _i32)

        @plsc.parallel_loop(0, C2, unroll=8)
        def _(e):
            wr[e, pl.ds(0, LANES)] = zero16
            wr[e, pl.ds(LANES, LANES)] = zero16

        _zero_acc(wr, msg_acc, sub)
        plsc.subcore_barrier()

        def idx_copies(t, b):
            base = (t * NSUB + sub) * C2
            return (
                pltpu.make_async_copy(
                    src_hbm.at[pl.ds(base, C2)], srcs.at[b], s_src[b]),
                pltpu.make_async_copy(
                    dst_hbm.at[pl.ds(base, C2)], dsts.at[b], s_dst[b]),
                pltpu.make_async_copy(
                    ex_hbm.at[pl.ds(4 * base, 4 * C2)], exvs.at[b], s_ex[b]),
            )

        def gather_copy(b):
            return pltpu.make_async_copy(hh_hbm.at[soffs.at[b]], hrs.at[b],
                                         s_hr[b])

        def prep_and_gather(b):
            @plsc.parallel_loop(0, C2 // LANES, unroll=4)
            def _(i):
                o = i * LANES
                soffs[b, pl.ds(o, LANES)] = \
                    srcs[b, pl.ds(o, LANES)] + core * N

            gather_copy(b).start()

        for d in idx_copies(0, 0):
            d.start()
        for d in idx_copies(0, 0):
            d.wait()
        prep_and_gather(0)
        for d in idx_copies(1, 1):
            d.start()

        @pl.loop(0, nch // 2)
        def _(to):
            for b in (0, 1):
                t = to * 2 + b
                q = 1 - b

                @pl.when(t + 1 < nch)
                def _():
                    for d in idx_copies(t + 1, q):
                        d.wait()
                    prep_and_gather(q)

                gather_copy(b).wait()

                @plsc.parallel_loop(0, C2, unroll=8)
                def _(e):
                    sp0 = plsc.load_gather(exvs.at[b], [h0idx + 4 * e])
                    sp1 = plsc.load_gather(exvs.at[b], [h1idx + 4 * e])
                    wr[e, pl.ds(0, LANES)] = hrs[b, e, pl.ds(0, LANES)] * sp0
                    wr[e, pl.ds(LANES, LANES)] = \
                        hrs[b, e, pl.ds(LANES, LANES)] * sp1

                pltpu.sync_copy(wr, msg_acc.at[dsts.at[b]], add=True)

                @pl.when(t + 2 < nch)
                def _():
                    for d in idx_copies(t + 2, b):
                        d.start()

        plsc.subcore_barrier()
        _dump_acc(msg_acc, msg_out, core, sub, N)

    return k


def _att_expand(att_l):
    att_flat = att_l.reshape(HD)
    cols = jnp.arange(F, dtype=_i32)[None, :]
    rows_h = (jnp.arange(HD, dtype=_i32) // F)[:, None]
    return jnp.where(cols == rows_h, att_flat[:, None], 0.0).astype(_f32)


def kernel(x, edge_index, W_in, b_in, W_gat, att_src, att_dst, b_gat,
           W_res, b_res, ln_g, ln_b, W_out, b_out):
    e_total = edge_index.shape[1] + N
    align = 32 * C1
    ep = ((e_total + align - 1) // align) * align
    pad = ep - e_total
    loops = jnp.arange(N, dtype=_i32)
    src = jnp.concatenate(
        [edge_index[0].astype(_i32), loops, jnp.zeros((pad,), _i32)])
    dst = jnp.concatenate(
        [edge_index[1].astype(_i32), loops, jnp.full((pad,), N, _i32)])

    sc_den = _sc_den_kernel(ep)
    sc_msg = _sc_msg_kernel(ep)

    h = _tc_in(x, W_in, b_in.reshape(1, HD))
    for l in range(L):
        asm = _att_expand(att_src[l])
        adm = _att_expand(att_dst[l])
        hh2, as_t, ad_t, res = _tc_pre(
            h, W_gat[l], W_res[l], b_res[l].reshape(1, HD), asm, adm)
        ex, den_flat = sc_den(as_t, ad_t, src, dst)
        msg_flat = sc_msg(hh2.reshape(2 * N, 32), ex, src, dst)
        h = _tc_post(msg_flat, den_flat, res, b_gat[l].reshape(1, HD),
                     ln_g[l].reshape(1, HD), ln_b[l].reshape(1, HD))
    return _tc_out(h, W_out, b_out.reshape(1, D_OUT))
